# baseline (speedup 1.0000x reference)
import jax
import jax.numpy as jnp
from jax import lax
from jax.experimental import pallas as pl
from jax.experimental.pallas import tpu as pltpu
from jax.experimental.pallas import tpu_sc as plsc

_MAX_TILES = 4
_HIDDEN = 1280
_PATCHES = 1025
_SPLIT = 6
_SC_BATCHES = 2
_SC_SLABS = _SC_BATCHES * _MAX_TILES
_QROWS = 256
_ROWS = 32
_NCHUNK = _QROWS // _ROWS
_NVEC = _HIDDEN // 16


def _tc_body(ids_ref, gate_ref, hid_ref, emb_ref, out_ref):
    g = jnp.tanh(gate_ref[0])
    out_ref[...] = hid_ref[...] + emb_ref[...] * g


def _sc_body(ids_hbm, scale_hbm, hid_hbm, table_hbm, tail_hbm,
             out_hbm, tailout_hbm,
             ids_sm, scale_sm, ev, bufs, tailbuf, evt,
             sem_misc, insem, outsem):
    core = lax.axis_index("core")
    sub = lax.axis_index("subcore")
    s = core * 16 + sub
    slab = s // 4
    q = s % 4
    gb = _SPLIT + slab // _MAX_TILES
    t = slab % _MAX_TILES

    pltpu.async_copy(ids_hbm, ids_sm, sem_misc).wait()
    pltpu.async_copy(scale_hbm, scale_sm, sem_misc).wait()

    idv = ids_sm[pl.ds(gb, 1)][0]
    base = idv * (_MAX_TILES * _HIDDEN) + t * _HIDDEN
    pltpu.async_copy(table_hbm.at[pl.ds(base, _HIDDEN)], ev, sem_misc).wait()

    sc = scale_sm[pl.ds(0, 1)][0]

    @pl.loop(0, _NVEC)
    def _(l):
        ev[pl.ds(l * 16, 16)] = ev[pl.ds(l * 16, 16)] * sc

    row0 = q * _QROWS

    def in_copy(c):
        return pltpu.make_async_copy(
            hid_hbm.at[_SPLIT * _MAX_TILES + slab,
                       pl.ds(row0 + c * _ROWS, _ROWS), :],
            bufs.at[c % 2], insem.at[c % 2])

    def out_copy(c):
        return pltpu.make_async_copy(
            bufs.at[c % 2],
            out_hbm.at[slab, pl.ds(row0 + c * _ROWS, _ROWS), :],
            outsem.at[c % 2])

    in_copy(0).start()

    @pl.loop(0, _NCHUNK)
    def _(c):
        in_copy(c).wait()

        @pl.when(c >= 1)
        def _():
            out_copy(c - 1).wait()

        @pl.when(c + 1 < _NCHUNK)
        def _():
            in_copy(c + 1).start()

        bufc = bufs.at[c % 2]

        @pl.loop(0, _ROWS)
        def _(r):
            row = bufc.at[r]
            for l in range(_NVEC):
                sl = pl.ds(l * 16, 16)
                row[sl] = row[sl] + ev[sl]

        out_copy(c).start()

    out_copy(_NCHUNK - 1).wait()

    @pl.when(s == 0)
    def _():
        pltpu.async_copy(tail_hbm, tailbuf, sem_misc).wait()
        for r in range(_SC_SLABS):
            bb = _SPLIT + r // _MAX_TILES
            tt = r % _MAX_TILES
            tbase = (ids_sm[pl.ds(bb, 1)][0] * (_MAX_TILES * _HIDDEN)
                     + tt * _HIDDEN)
            pltpu.async_copy(table_hbm.at[pl.ds(tbase, _HIDDEN)], evt,
                             sem_misc).wait()

            @pl.loop(0, _NVEC)
            def _(l):
                sl = pl.ds(l * 16, 16)
                tailbuf[r, sl] = tailbuf[r, sl] + evt[sl] * sc

        pltpu.async_copy(tailbuf, tailout_hbm, sem_misc).wait()


def kernel(hidden_state, aspect_ratio_ids, embedding_table, gate):
    batch = hidden_state.shape[0]
    ids = aspect_ratio_ids.astype(jnp.int32)
    scale = jnp.tanh(gate)
    table4 = embedding_table.reshape(-1, _MAX_TILES, 1, _HIDDEN)
    table1d = embedding_table.reshape(-1)

    tc_out = pl.pallas_call(
        _tc_body,
        grid_spec=pltpu.PrefetchScalarGridSpec(
            num_scalar_prefetch=2,
            grid=(_SPLIT, _MAX_TILES),
            in_specs=[
                pl.BlockSpec(
                    (1, 1, _PATCHES, _HIDDEN),
                    lambda b, t, ids, gate: (b, t, 0, 0),
                ),
                pl.BlockSpec(
                    (1, 1, 1, _HIDDEN),
                    lambda b, t, ids, gate: (ids[b], t, 0, 0),
                ),
            ],
            out_specs=pl.BlockSpec(
                (1, 1, _PATCHES, _HIDDEN),
                lambda b, t, ids, gate: (b, t, 0, 0),
            ),
        ),
        out_shape=jax.ShapeDtypeStruct(hidden_state.shape, hidden_state.dtype),
        compiler_params=pltpu.CompilerParams(
            dimension_semantics=("parallel", "parallel"),
        ),
    )(ids, gate, hidden_state, table4)

    hid3 = hidden_state.reshape(batch * _MAX_TILES, _PATCHES, _HIDDEN)
    tail_sc = lax.slice(
        hidden_state, (_SPLIT, 0, _PATCHES - 1, 0),
        (batch, _MAX_TILES, _PATCHES, _HIDDEN),
    ).reshape(_SC_SLABS, _HIDDEN)

    mesh = plsc.VectorSubcoreMesh(core_axis_name="core",
                                  subcore_axis_name="subcore")
    sc_call = pl.kernel(
        _sc_body,
        out_type=[
            jax.ShapeDtypeStruct((_SC_SLABS, _PATCHES, _HIDDEN),
                                 hidden_state.dtype),
            jax.ShapeDtypeStruct(tail_sc.shape, tail_sc.dtype),
        ],
        mesh=mesh,
        scratch_types=[
            pltpu.VMEM((batch,), jnp.int32),
            pltpu.VMEM((1,), jnp.float32),
            pltpu.VMEM((_HIDDEN,), jnp.float32),
            pltpu.VMEM((2, _ROWS, _HIDDEN), jnp.float32),
            pltpu.VMEM((_SC_SLABS, _HIDDEN), jnp.float32),
            pltpu.VMEM((_HIDDEN,), jnp.float32),
            pltpu.SemaphoreType.DMA,
            pltpu.SemaphoreType.DMA((2,)),
            pltpu.SemaphoreType.DMA((2,)),
        ],
    )
    sc_main, sc_tail = sc_call(ids, scale, hid3, table1d, tail_sc)

    sc_full = lax.dynamic_update_slice(
        sc_main.reshape(_SC_BATCHES, _MAX_TILES, _PATCHES, _HIDDEN),
        sc_tail.reshape(_SC_BATCHES, _MAX_TILES, 1, _HIDDEN),
        (0, 0, _PATCHES - 1, 0))
    out = lax.dynamic_update_slice(tc_out, sc_full, (_SPLIT, 0, 0, 0))
    return out


# speedup vs baseline: 2.3683x; 2.3683x over previous
import jax
import jax.numpy as jnp
from jax.experimental import pallas as pl
from jax.experimental.pallas import tpu as pltpu

_MAX_TILES = 4
_HIDDEN = 1280
_PATCHES = 1025


def _body(ids_ref, gate_ref, hid_ref, emb_ref, out_ref):
    g = jnp.tanh(gate_ref[0])
    out_ref[...] = hid_ref[...] + emb_ref[...] * g


def kernel(hidden_state, aspect_ratio_ids, embedding_table, gate):
    batch = hidden_state.shape[0]
    ids = aspect_ratio_ids.astype(jnp.int32)
    table = embedding_table.reshape(-1, _MAX_TILES, 1, _HIDDEN)
    grid = (batch, _MAX_TILES // 2)

    out = pl.pallas_call(
        _body,
        grid_spec=pltpu.PrefetchScalarGridSpec(
            num_scalar_prefetch=2,
            grid=grid,
            in_specs=[
                pl.BlockSpec(
                    (1, 2, _PATCHES, _HIDDEN),
                    lambda b, t, ids, gate: (b, t, 0, 0),
                ),
                pl.BlockSpec(
                    (1, 2, 1, _HIDDEN),
                    lambda b, t, ids, gate: (ids[b], t, 0, 0),
                ),
            ],
            out_specs=pl.BlockSpec(
                (1, 2, _PATCHES, _HIDDEN),
                lambda b, t, ids, gate: (b, t, 0, 0),
            ),
        ),
        out_shape=jax.ShapeDtypeStruct(hidden_state.shape, hidden_state.dtype),
        compiler_params=pltpu.CompilerParams(
            dimension_semantics=("parallel", "parallel"),
            vmem_limit_bytes=100 * 1024 * 1024,
        ),
    )(ids, gate, hidden_state, table)
    return out


# speedup vs baseline: 2.3683x; 1.0000x over previous
import jax
import jax.numpy as jnp
from jax.experimental import pallas as pl
from jax.experimental.pallas import tpu as pltpu

_MAX_TILES = 4
_HIDDEN = 1280
_PATCHES = 1025


def _body(ids_ref, gate_ref, hid_ref, emb_ref, out_ref):
    g = jnp.tanh(gate_ref[0])
    out_ref[...] = hid_ref[...] + emb_ref[...] * g


def kernel(hidden_state, aspect_ratio_ids, embedding_table, gate):
    batch = hidden_state.shape[0]
    ids = aspect_ratio_ids.astype(jnp.int32)
    table = embedding_table.reshape(-1, _MAX_TILES, 1, _HIDDEN)
    grid = (batch, _MAX_TILES // 2)

    out = pl.pallas_call(
        _body,
        grid_spec=pltpu.PrefetchScalarGridSpec(
            num_scalar_prefetch=2,
            grid=grid,
            in_specs=[
                pl.BlockSpec(
                    (1, 2, _PATCHES, _HIDDEN),
                    lambda b, t, ids, gate: (b, t, 0, 0),
                ),
                pl.BlockSpec(
                    (1, 2, 1, _HIDDEN),
                    lambda b, t, ids, gate: (ids[b], t, 0, 0),
                ),
            ],
            out_specs=pl.BlockSpec(
                (1, 2, _PATCHES, _HIDDEN),
                lambda b, t, ids, gate: (b, t, 0, 0),
            ),
        ),
        out_shape=jax.ShapeDtypeStruct(hidden_state.shape, hidden_state.dtype),
        compiler_params=pltpu.CompilerParams(
            dimension_semantics=("arbitrary", "arbitrary"),
            vmem_limit_bytes=100 * 1024 * 1024,
        ),
    )(ids, gate, hidden_state, table)
    return out


# speedup vs baseline: 2.3691x; 1.0004x over previous
import jax
import jax.numpy as jnp
from jax.experimental import pallas as pl
from jax.experimental.pallas import tpu as pltpu

_MAX_TILES = 4
_HIDDEN = 1280
_PATCHES = 1025


def _body(ids_ref, gate_ref, hid_ref, emb_ref, out_ref):
    g = jnp.tanh(gate_ref[0])
    out_ref[...] = hid_ref[...] + emb_ref[...] * g


def kernel(hidden_state, aspect_ratio_ids, embedding_table, gate):
    batch = hidden_state.shape[0]
    ids = aspect_ratio_ids.astype(jnp.int32)
    table = embedding_table.reshape(-1, _MAX_TILES, 1, _HIDDEN)
    grid = (batch, _MAX_TILES // 2)

    out = pl.pallas_call(
        _body,
        grid_spec=pltpu.PrefetchScalarGridSpec(
            num_scalar_prefetch=2,
            grid=grid,
            in_specs=[
                pl.BlockSpec(
                    (1, 2, _PATCHES, _HIDDEN),
                    lambda b, t, ids, gate: (b, t, 0, 0),
                ),
                pl.BlockSpec(
                    (1, 2, 1, _HIDDEN),
                    lambda b, t, ids, gate: (ids[b], t, 0, 0),
                ),
            ],
            out_specs=pl.BlockSpec(
                (1, 2, _PATCHES, _HIDDEN),
                lambda b, t, ids, gate: (b, t, 0, 0),
            ),
        ),
        out_shape=jax.ShapeDtypeStruct(hidden_state.shape, hidden_state.dtype),
        compiler_params=pltpu.CompilerParams(
            dimension_semantics=("parallel", "parallel"),
            vmem_limit_bytes=100 * 1024 * 1024,
        ),
    )(ids, gate, hidden_state, table)
    return out
